# Initial kernel scaffold; baseline (speedup 1.0000x reference)
import jax
import jax.numpy as jnp
from jax.experimental import pallas as pl


def kernel(x, Wg, W_in, V_in, W_out, Ws_in, Vs_in, Ws_out):
    raise NotImplementedError("write your pallas kernel here")


# speedup vs baseline: 1.2647x; 1.2647x over previous
import functools

import jax
import jax.numpy as jnp
from jax import lax
from jax.experimental import pallas as pl
from jax.experimental.pallas import tpu as pltpu
from jax.experimental.pallas import tpu_sc as plsc

S = 2048
D = 2048
INTER = 1024
E = 8
K = 2
SH = 2

BT = 256
P = S * K + E * BT
NB = P // BT
NC, NS = 2, 16
NW = NC * NS


def _route_body(x_ref, wg_ref, s0_ref, s1_ref, w0_ref, w1_ref, eb_ref):
    g = jax.nn.sigmoid(
        lax.dot_general(x_ref[...], wg_ref[...],
                        (((1,), (1,)), ((), ())),
                        preferred_element_type=jnp.float32))
    idx = lax.broadcasted_iota(jnp.int32, (S, E), 1)
    m0 = jnp.max(g, axis=1, keepdims=True)
    i0 = jnp.min(jnp.where(g == m0, idx, E), axis=1, keepdims=True)
    oh0 = idx == i0
    gm = jnp.where(oh0, -jnp.inf, g)
    m1 = jnp.max(gm, axis=1, keepdims=True)
    i1 = jnp.min(jnp.where(gm == m1, idx, E), axis=1, keepdims=True)
    oh1 = idx == i1
    wsum = m0 + m1
    w0_ref[...] = jnp.broadcast_to(m0 / wsum, (S, 16))
    w1_ref[...] = jnp.broadcast_to(m1 / wsum, (S, 16))

    mm = oh0.astype(jnp.float32) + oh1.astype(jnp.float32)
    r = lax.broadcasted_iota(jnp.int32, (S, S), 0)
    c = lax.broadcasted_iota(jnp.int32, (S, S), 1)
    lt = (r > c).astype(jnp.float32)
    rank = lax.dot_general(lt, mm, (((1,), (0,)), ((), ())),
                           preferred_element_type=jnp.float32)

    cnt = jnp.sum(mm, axis=0, keepdims=True)
    used = jnp.floor((cnt + (BT - 1)) / BT)
    er = lax.broadcasted_iota(jnp.int32, (E, E), 0)
    ec = lax.broadcasted_iota(jnp.int32, (E, E), 1)
    ut = (er <= ec).astype(jnp.float32)
    cum_incl = lax.dot_general(used, ut, (((1,), (0,)), ((), ())),
                               preferred_element_type=jnp.float32)
    base = (cum_incl - used) * BT

    slot = rank + base
    s0_ref[...] = jnp.sum(jnp.where(oh0, slot, 0.0), axis=1,
                          keepdims=True).astype(jnp.int32)
    s1_ref[...] = jnp.sum(jnp.where(oh1, slot, 0.0), axis=1,
                          keepdims=True).astype(jnp.int32)

    b_i = lax.broadcasted_iota(jnp.int32, (NB, E), 0)
    cum_i = jnp.broadcast_to(cum_incl.astype(jnp.int32), (NB, E))
    eb_ref[...] = jnp.sum((b_i >= cum_i).astype(jnp.int32), axis=1,
                          keepdims=True)


def _route(xs, Wg, interpret=False):
    return pl.pallas_call(
        _route_body,
        out_shape=(
            jax.ShapeDtypeStruct((S, 1), jnp.int32),
            jax.ShapeDtypeStruct((S, 1), jnp.int32),
            jax.ShapeDtypeStruct((S, 16), jnp.float32),
            jax.ShapeDtypeStruct((S, 16), jnp.float32),
            jax.ShapeDtypeStruct((NB, 1), jnp.int32),
        ),
        interpret=interpret,
    )(xs, Wg)


_DCHUNK = 16
_DPW = S // NW


def _dispatch_body(s0_hbm, s1_hbm, x_hbm, xg_hbm, idx0_v, idx1_v, rows_v, sem):
    wid = lax.axis_index("s") * NC + lax.axis_index("c")
    base_w = wid * _DPW
    for ci in range(_DPW // _DCHUNK):
        base = base_w + ci * _DCHUNK
        pltpu.sync_copy(s0_hbm.at[pl.ds(base, _DCHUNK)], idx0_v)
        pltpu.sync_copy(s1_hbm.at[pl.ds(base, _DCHUNK)], idx1_v)
        pltpu.sync_copy(x_hbm.at[pl.ds(base, _DCHUNK)], rows_v)
        pltpu.async_copy(rows_v, xg_hbm.at[idx0_v], sem).wait()
        pltpu.async_copy(rows_v, xg_hbm.at[idx1_v], sem).wait()


def _dispatch(s0, s1, xs):
    mesh = plsc.VectorSubcoreMesh(core_axis_name="c", subcore_axis_name="s")
    return pl.kernel(
        _dispatch_body,
        out_type=jax.ShapeDtypeStruct((P, D), jnp.float32),
        mesh=mesh,
        scratch_types=[
            pltpu.VMEM((_DCHUNK,), jnp.int32),
            pltpu.VMEM((_DCHUNK,), jnp.int32),
            pltpu.VMEM((_DCHUNK, D), jnp.float32),
            pltpu.SemaphoreType.DMA,
        ],
    )(s0, s1, xs)


def _f1_body(eb_ref, xg_ref, win_ref, vin_ref, h_ref):
    b = pl.program_id(0)

    @pl.when(eb_ref[b] < E)
    def _():
        xb = xg_ref[...]
        a = lax.dot_general(xb, win_ref[0], (((1,), (1,)), ((), ())),
                            preferred_element_type=jnp.float32)
        bq = lax.dot_general(xb, vin_ref[0], (((1,), (1,)), ((), ())),
                             preferred_element_type=jnp.float32)
        h_ref[...] = (a * jax.nn.sigmoid(a)) * bq


def _f1(eb, xg, W_in, V_in, interpret=False):
    grid_spec = pltpu.PrefetchScalarGridSpec(
        num_scalar_prefetch=1,
        grid=(NB,),
        in_specs=[
            pl.BlockSpec((BT, D), lambda b, eb: (b, 0)),
            pl.BlockSpec((1, INTER, D),
                         lambda b, eb: (jnp.minimum(eb[b], E - 1), 0, 0)),
            pl.BlockSpec((1, INTER, D),
                         lambda b, eb: (jnp.minimum(eb[b], E - 1), 0, 0)),
        ],
        out_specs=pl.BlockSpec((BT, INTER), lambda b, eb: (b, 0)),
    )
    return pl.pallas_call(
        _f1_body,
        grid_spec=grid_spec,
        out_shape=jax.ShapeDtypeStruct((P, INTER), jnp.float32),
        interpret=interpret,
    )(eb, xg, W_in, V_in)


def _f2_body(eb_ref, h_ref, wout_ref, y_ref):
    b = pl.program_id(0)

    @pl.when(eb_ref[b] < E)
    def _():
        y_ref[...] = lax.dot_general(h_ref[...], wout_ref[0],
                                     (((1,), (1,)), ((), ())),
                                     preferred_element_type=jnp.float32)


def _f2(eb, H, W_out, interpret=False):
    grid_spec = pltpu.PrefetchScalarGridSpec(
        num_scalar_prefetch=1,
        grid=(NB,),
        in_specs=[
            pl.BlockSpec((BT, INTER), lambda b, eb: (b, 0)),
            pl.BlockSpec((1, D, INTER),
                         lambda b, eb: (jnp.minimum(eb[b], E - 1), 0, 0)),
        ],
        out_specs=pl.BlockSpec((BT, D), lambda b, eb: (b, 0)),
    )
    return pl.pallas_call(
        _f2_body,
        grid_spec=grid_spec,
        out_shape=jax.ShapeDtypeStruct((P, D), jnp.float32),
        interpret=interpret,
    )(eb, H, W_out)


BS = 256
IB = 1024


def _f3_body(x_ref, win_ref, vin_ref, h_ref):
    xb = x_ref[...]
    a = lax.dot_general(xb, win_ref[...], (((1,), (1,)), ((), ())),
                        preferred_element_type=jnp.float32)
    bq = lax.dot_general(xb, vin_ref[...], (((1,), (1,)), ((), ())),
                         preferred_element_type=jnp.float32)
    h_ref[...] = (a * jax.nn.sigmoid(a)) * bq


def _f3(xs, Ws_in, Vs_in, interpret=False):
    ni = SH * INTER // IB
    return pl.pallas_call(
        _f3_body,
        grid=(ni, S // BS),
        in_specs=[
            pl.BlockSpec((BS, D), lambda i, s: (s, 0)),
            pl.BlockSpec((IB, D), lambda i, s: (i, 0)),
            pl.BlockSpec((IB, D), lambda i, s: (i, 0)),
        ],
        out_specs=pl.BlockSpec((BS, IB), lambda i, s: (s, i)),
        out_shape=jax.ShapeDtypeStruct((S, SH * INTER), jnp.float32),
        interpret=interpret,
    )(xs, Ws_in, Vs_in)


def _f4_body(h_ref, wout_ref, y_ref):
    i = pl.program_id(1)
    part = lax.dot_general(h_ref[...], wout_ref[...],
                           (((1,), (1,)), ((), ())),
                           preferred_element_type=jnp.float32)

    @pl.when(i == 0)
    def _():
        y_ref[...] = part

    @pl.when(i > 0)
    def _():
        y_ref[...] += part


def _f4(Hs, Ws_out, interpret=False):
    ni = SH * INTER // IB
    return pl.pallas_call(
        _f4_body,
        grid=(S // BS, ni),
        in_specs=[
            pl.BlockSpec((BS, IB), lambda s, i: (s, i)),
            pl.BlockSpec((D, IB), lambda s, i: (0, i)),
        ],
        out_specs=pl.BlockSpec((BS, D), lambda s, i: (s, 0)),
        out_shape=jax.ShapeDtypeStruct((S, D), jnp.float32),
        interpret=interpret,
    )(Hs, Ws_out)


_CCHUNK = 16
_CPW = S // NW


def _combine_body(s0_hbm, s1_hbm, w0_hbm, w1_hbm, ysh_hbm, y_hbm, out_hbm,
                  idx0_v, idx1_v, w0_v, w1_v, acc_v, y0_v, y1_v, sem):
    wid = lax.axis_index("s") * NC + lax.axis_index("c")
    base_w = wid * _CPW
    for ci in range(_CPW // _CCHUNK):
        base = base_w + ci * _CCHUNK
        pltpu.sync_copy(s0_hbm.at[pl.ds(base, _CCHUNK)], idx0_v)
        pltpu.sync_copy(s1_hbm.at[pl.ds(base, _CCHUNK)], idx1_v)
        pltpu.sync_copy(w0_hbm.at[pl.ds(base, _CCHUNK)], w0_v)
        pltpu.sync_copy(w1_hbm.at[pl.ds(base, _CCHUNK)], w1_v)
        pltpu.sync_copy(ysh_hbm.at[pl.ds(base, _CCHUNK)], acc_v)
        pltpu.async_copy(y_hbm.at[idx0_v], y0_v, sem).wait()
        pltpu.async_copy(y_hbm.at[idx1_v], y1_v, sem).wait()
        for ti in range(_CCHUNK):
            w0b = w0_v[ti, :]
            w1b = w1_v[ti, :]

            def jbody(j, _, ti=ti, w0b=w0b, w1b=w1b):
                off = j * 16
                a = (acc_v[ti, pl.ds(off, 16)]
                     + w0b * y0_v[ti, pl.ds(off, 16)]
                     + w1b * y1_v[ti, pl.ds(off, 16)])
                acc_v[ti, pl.ds(off, 16)] = a
                return 0

            lax.fori_loop(0, D // 16, jbody, 0)
        pltpu.sync_copy(acc_v, out_hbm.at[pl.ds(base, _CCHUNK)])


def _combine(s0, s1, w0, w1, ysh, y):
    mesh = plsc.VectorSubcoreMesh(core_axis_name="c", subcore_axis_name="s")
    return pl.kernel(
        _combine_body,
        out_type=jax.ShapeDtypeStruct((S, D), jnp.float32),
        mesh=mesh,
        scratch_types=[
            pltpu.VMEM((_CCHUNK,), jnp.int32),
            pltpu.VMEM((_CCHUNK,), jnp.int32),
            pltpu.VMEM((_CCHUNK, 16), jnp.float32),
            pltpu.VMEM((_CCHUNK, 16), jnp.float32),
            pltpu.VMEM((_CCHUNK, D), jnp.float32),
            pltpu.VMEM((_CCHUNK, D), jnp.float32),
            pltpu.VMEM((_CCHUNK, D), jnp.float32),
            pltpu.SemaphoreType.DMA,
        ],
    )(s0, s1, w0, w1, ysh, y)


def kernel(x, Wg, W_in, V_in, W_out, Ws_in, Vs_in, Ws_out):
    xs = x.reshape(S, D)
    s0, s1, w0, w1, eb = _route(xs, Wg)
    s0r, s1r = s0.reshape(S), s1.reshape(S)
    xg = _dispatch(s0r, s1r, xs)
    H = _f1(eb.reshape(NB), xg, W_in, V_in)
    y = _f2(eb.reshape(NB), H, W_out)
    Hs = _f3(xs, Ws_in, Vs_in)
    ysh = _f4(Hs, Ws_out)
    out = _combine(s0r, s1r, w0, w1, ysh, y)
    return out.reshape(1, S, D)


# speedup vs baseline: 1.3716x; 1.0846x over previous
import functools

import jax
import jax.numpy as jnp
from jax import lax
from jax.experimental import pallas as pl
from jax.experimental.pallas import tpu as pltpu
from jax.experimental.pallas import tpu_sc as plsc

S = 2048
D = 2048
INTER = 1024
E = 8
K = 2
SH = 2

BT = 256
P = S * K + E * BT
NB = P // BT
NC, NS = 2, 16
NW = NC * NS


def _route_body(x_ref, wg_ref, s0_ref, s1_ref, w0_ref, w1_ref, eb_ref):
    g = jax.nn.sigmoid(
        lax.dot_general(x_ref[...], wg_ref[...],
                        (((1,), (1,)), ((), ())),
                        preferred_element_type=jnp.float32))
    idx = lax.broadcasted_iota(jnp.int32, (S, E), 1)
    m0 = jnp.max(g, axis=1, keepdims=True)
    i0 = jnp.min(jnp.where(g == m0, idx, E), axis=1, keepdims=True)
    oh0 = idx == i0
    gm = jnp.where(oh0, -jnp.inf, g)
    m1 = jnp.max(gm, axis=1, keepdims=True)
    i1 = jnp.min(jnp.where(gm == m1, idx, E), axis=1, keepdims=True)
    oh1 = idx == i1
    wsum = m0 + m1
    w0_ref[...] = jnp.broadcast_to(m0 / wsum, (S, 128))
    w1_ref[...] = jnp.broadcast_to(m1 / wsum, (S, 128))

    mm = oh0.astype(jnp.float32) + oh1.astype(jnp.float32)
    r = lax.broadcasted_iota(jnp.int32, (S, S), 0)
    c = lax.broadcasted_iota(jnp.int32, (S, S), 1)
    lt = (r > c).astype(jnp.float32)
    rank = lax.dot_general(lt, mm, (((1,), (0,)), ((), ())),
                           preferred_element_type=jnp.float32)

    cnt = jnp.sum(mm, axis=0, keepdims=True)
    used = jnp.floor((cnt + (BT - 1)) / BT)
    er = lax.broadcasted_iota(jnp.int32, (E, E), 0)
    ec = lax.broadcasted_iota(jnp.int32, (E, E), 1)
    ut = (er <= ec).astype(jnp.float32)
    cum_incl = lax.dot_general(used, ut, (((1,), (0,)), ((), ())),
                               preferred_element_type=jnp.float32)
    base = (cum_incl - used) * BT

    slot = rank + base
    s0_ref[...] = jnp.sum(jnp.where(oh0, slot, 0.0), axis=1,
                          keepdims=True).astype(jnp.int32)
    s1_ref[...] = jnp.sum(jnp.where(oh1, slot, 0.0), axis=1,
                          keepdims=True).astype(jnp.int32)

    b_i = lax.broadcasted_iota(jnp.int32, (NB, E), 0)
    cum_i = jnp.broadcast_to(cum_incl.astype(jnp.int32), (NB, E))
    eb_ref[...] = jnp.sum((b_i >= cum_i).astype(jnp.int32), axis=1,
                          keepdims=True)


def _route(xs, Wg, interpret=False):
    return pl.pallas_call(
        _route_body,
        out_shape=(
            jax.ShapeDtypeStruct((S, 1), jnp.int32),
            jax.ShapeDtypeStruct((S, 1), jnp.int32),
            jax.ShapeDtypeStruct((S, 128), jnp.float32),
            jax.ShapeDtypeStruct((S, 128), jnp.float32),
            jax.ShapeDtypeStruct((NB, 1), jnp.int32),
        ),
        interpret=interpret,
    )(xs, Wg)


_DCHUNK = 32
_DPW = S // NW


def _dispatch_body(s0_hbm, s1_hbm, w0_hbm, w1_hbm, x_hbm, xg_hbm, wsl_hbm,
                   idx0_v, idx1_v, w0_v, w1_v, rows_v, sem):
    wid = lax.axis_index("s") * NC + lax.axis_index("c")
    base_w = wid * _DPW
    for ci in range(_DPW // _DCHUNK):
        base = base_w + ci * _DCHUNK
        pltpu.sync_copy(s0_hbm.at[pl.ds(base, _DCHUNK)], idx0_v)
        pltpu.sync_copy(s1_hbm.at[pl.ds(base, _DCHUNK)], idx1_v)
        pltpu.sync_copy(w0_hbm.at[pl.ds(base, _DCHUNK)], w0_v)
        pltpu.sync_copy(w1_hbm.at[pl.ds(base, _DCHUNK)], w1_v)
        pltpu.sync_copy(x_hbm.at[pl.ds(base, _DCHUNK)], rows_v)
        c0 = pltpu.async_copy(rows_v, xg_hbm.at[idx0_v], sem)
        c1 = pltpu.async_copy(rows_v, xg_hbm.at[idx1_v], sem)
        c2 = pltpu.async_copy(w0_v, wsl_hbm.at[idx0_v], sem)
        c3 = pltpu.async_copy(w1_v, wsl_hbm.at[idx1_v], sem)
        c0.wait(); c1.wait(); c2.wait(); c3.wait()


def _dispatch(s0, s1, w0, w1, xs):
    mesh = plsc.VectorSubcoreMesh(core_axis_name="c", subcore_axis_name="s")
    return pl.kernel(
        _dispatch_body,
        out_type=(
            jax.ShapeDtypeStruct((P, D), jnp.float32),
            jax.ShapeDtypeStruct((P, 128), jnp.float32),
        ),
        mesh=mesh,
        scratch_types=[
            pltpu.VMEM((_DCHUNK,), jnp.int32),
            pltpu.VMEM((_DCHUNK,), jnp.int32),
            pltpu.VMEM((_DCHUNK, 128), jnp.float32),
            pltpu.VMEM((_DCHUNK, 128), jnp.float32),
            pltpu.VMEM((_DCHUNK, D), jnp.float32),
            pltpu.SemaphoreType.DMA,
        ],
    )(s0, s1, w0, w1, xs)


def _f1_body(eb_ref, xg_ref, win_ref, vin_ref, wsl_ref, h_ref):
    b = pl.program_id(0)

    @pl.when(eb_ref[b] < E)
    def _():
        xb = xg_ref[...]
        a = lax.dot_general(xb, win_ref[0], (((1,), (1,)), ((), ())),
                            preferred_element_type=jnp.float32)
        bq = lax.dot_general(xb, vin_ref[0], (((1,), (1,)), ((), ())),
                             preferred_element_type=jnp.float32)
        h_ref[...] = (a * jax.nn.sigmoid(a)) * bq * wsl_ref[:, :1]


def _f1(eb, xg, W_in, V_in, wsl, interpret=False):
    grid_spec = pltpu.PrefetchScalarGridSpec(
        num_scalar_prefetch=1,
        grid=(NB,),
        in_specs=[
            pl.BlockSpec((BT, D), lambda b, eb: (b, 0)),
            pl.BlockSpec((1, INTER, D),
                         lambda b, eb: (jnp.minimum(eb[b], E - 1), 0, 0)),
            pl.BlockSpec((1, INTER, D),
                         lambda b, eb: (jnp.minimum(eb[b], E - 1), 0, 0)),
            pl.BlockSpec((BT, 128), lambda b, eb: (b, 0)),
        ],
        out_specs=pl.BlockSpec((BT, INTER), lambda b, eb: (b, 0)),
    )
    return pl.pallas_call(
        _f1_body,
        grid_spec=grid_spec,
        out_shape=jax.ShapeDtypeStruct((P, INTER), jnp.float32),
        interpret=interpret,
    )(eb, xg, W_in, V_in, wsl)


def _f2_body(eb_ref, h_ref, wout_ref, y_ref):
    b = pl.program_id(0)

    @pl.when(eb_ref[b] < E)
    def _():
        y_ref[...] = lax.dot_general(h_ref[...], wout_ref[0],
                                     (((1,), (1,)), ((), ())),
                                     preferred_element_type=jnp.float32)


def _f2(eb, H, W_out, interpret=False):
    grid_spec = pltpu.PrefetchScalarGridSpec(
        num_scalar_prefetch=1,
        grid=(NB,),
        in_specs=[
            pl.BlockSpec((BT, INTER), lambda b, eb: (b, 0)),
            pl.BlockSpec((1, D, INTER),
                         lambda b, eb: (jnp.minimum(eb[b], E - 1), 0, 0)),
        ],
        out_specs=pl.BlockSpec((BT, D), lambda b, eb: (b, 0)),
    )
    return pl.pallas_call(
        _f2_body,
        grid_spec=grid_spec,
        out_shape=jax.ShapeDtypeStruct((P, D), jnp.float32),
        interpret=interpret,
    )(eb, H, W_out)


BS = 256
IB = 1024


def _f3_body(x_ref, win_ref, vin_ref, h_ref):
    xb = x_ref[...]
    a = lax.dot_general(xb, win_ref[...], (((1,), (1,)), ((), ())),
                        preferred_element_type=jnp.float32)
    bq = lax.dot_general(xb, vin_ref[...], (((1,), (1,)), ((), ())),
                         preferred_element_type=jnp.float32)
    h_ref[...] = (a * jax.nn.sigmoid(a)) * bq


def _f3(xs, Ws_in, Vs_in, interpret=False):
    ni = SH * INTER // IB
    return pl.pallas_call(
        _f3_body,
        grid=(ni, S // BS),
        in_specs=[
            pl.BlockSpec((BS, D), lambda i, s: (s, 0)),
            pl.BlockSpec((IB, D), lambda i, s: (i, 0)),
            pl.BlockSpec((IB, D), lambda i, s: (i, 0)),
        ],
        out_specs=pl.BlockSpec((BS, IB), lambda i, s: (s, i)),
        out_shape=jax.ShapeDtypeStruct((S, SH * INTER), jnp.float32),
        interpret=interpret,
    )(xs, Ws_in, Vs_in)


def _f4_body(h_ref, wout_ref, y_ref):
    i = pl.program_id(1)
    part = lax.dot_general(h_ref[...], wout_ref[...],
                           (((1,), (1,)), ((), ())),
                           preferred_element_type=jnp.float32)

    @pl.when(i == 0)
    def _():
        y_ref[...] = part

    @pl.when(i > 0)
    def _():
        y_ref[...] += part


def _f4(Hs, Ws_out, interpret=False):
    ni = SH * INTER // IB
    return pl.pallas_call(
        _f4_body,
        grid=(S // BS, ni),
        in_specs=[
            pl.BlockSpec((BS, IB), lambda s, i: (s, i)),
            pl.BlockSpec((D, IB), lambda s, i: (0, i)),
        ],
        out_specs=pl.BlockSpec((BS, D), lambda s, i: (s, 0)),
        out_shape=jax.ShapeDtypeStruct((S, D), jnp.float32),
        interpret=interpret,
    )(Hs, Ws_out)


_CCHUNK = 16
_CPW = S // NW


def _combine_body(s0_hbm, s1_hbm, ysh_hbm, y_hbm, out_hbm,
                  idx0_v, idx1_v, acc_v, y0_v, y1_v, sem):
    wid = lax.axis_index("s") * NC + lax.axis_index("c")
    base_w = wid * _CPW
    for ci in range(_CPW // _CCHUNK):
        base = base_w + ci * _CCHUNK
        pltpu.sync_copy(s0_hbm.at[pl.ds(base, _CCHUNK)], idx0_v)
        pltpu.sync_copy(s1_hbm.at[pl.ds(base, _CCHUNK)], idx1_v)
        c0 = pltpu.async_copy(y_hbm.at[idx0_v], y0_v, sem)
        c1 = pltpu.async_copy(y_hbm.at[idx1_v], y1_v, sem)
        pltpu.sync_copy(ysh_hbm.at[pl.ds(base, _CCHUNK)], acc_v)
        c0.wait()
        c1.wait()
        for ti in range(_CCHUNK):
            def jbody(j, _, ti=ti):
                off = j * 16
                acc_v[ti, pl.ds(off, 16)] = (acc_v[ti, pl.ds(off, 16)]
                                             + y0_v[ti, pl.ds(off, 16)]
                                             + y1_v[ti, pl.ds(off, 16)])
                return 0

            lax.fori_loop(0, D // 16, jbody, 0, unroll=4)
        pltpu.sync_copy(acc_v, out_hbm.at[pl.ds(base, _CCHUNK)])


def _combine(s0, s1, ysh, y):
    mesh = plsc.VectorSubcoreMesh(core_axis_name="c", subcore_axis_name="s")
    return pl.kernel(
        _combine_body,
        out_type=jax.ShapeDtypeStruct((S, D), jnp.float32),
        mesh=mesh,
        scratch_types=[
            pltpu.VMEM((_CCHUNK,), jnp.int32),
            pltpu.VMEM((_CCHUNK,), jnp.int32),
            pltpu.VMEM((_CCHUNK, D), jnp.float32),
            pltpu.VMEM((_CCHUNK, D), jnp.float32),
            pltpu.VMEM((_CCHUNK, D), jnp.float32),
            pltpu.SemaphoreType.DMA,
        ],
    )(s0, s1, ysh, y)


def kernel(x, Wg, W_in, V_in, W_out, Ws_in, Vs_in, Ws_out):
    xs = x.reshape(S, D)
    s0, s1, w0, w1, eb = _route(xs, Wg)
    s0r, s1r = s0.reshape(S), s1.reshape(S)
    xg, wsl = _dispatch(s0r, s1r, w0, w1, xs)
    H = _f1(eb.reshape(NB), xg, W_in, V_in, wsl)
    y = _f2(eb.reshape(NB), H, W_out)
    Hs = _f3(xs, Ws_in, Vs_in)
    ysh = _f4(Hs, Ws_out)
    out = _combine(s0r, s1r, ysh, y)
    return out.reshape(1, S, D)
